# baseline (speedup 1.0000x reference)
import functools

import jax
import jax.numpy as jnp
from jax import lax
from jax.experimental import pallas as pl
from jax.experimental.pallas import tpu as pltpu
from jax.experimental.pallas import tpu_sc as plsc

VOCAB = 1000000
EMB = 64
B = 16384
LANE = 128
NCOL = (VOCAB + LANE - 1) // LANE

_info = plsc.get_sparse_core_info()
NC, NS, L = _info.num_cores, _info.num_subcores, _info.num_lanes
NW = NC * NS

NCHUNK = 4
CHUNK = B // NCHUNK
WCOL = 4
WIN = WCOL * LANE
NEP = 62
MAXH = 800
LISTC = 1056
SENT = 0x3FFFFFFF


@functools.partial(
    pl.kernel,
    mesh=plsc.VectorSubcoreMesh(core_axis_name="c", subcore_axis_name="s"),
    out_type=jax.ShapeDtypeStruct((B * EMB,), jnp.float32),
    scratch_types=[
        pltpu.VMEM((CHUNK,), jnp.int32),
        pltpu.VMEM((LISTC,), jnp.int32),
        pltpu.VMEM((LISTC,), jnp.int32),
        pltpu.VMEM((2, EMB, WIN), jnp.float32),
        pltpu.VMEM((MAXH // 2, 2 * EMB), jnp.float32),
        pltpu.SemaphoreType.DMA,
        pltpu.SemaphoreType.DMA,
        pltpu.SemaphoreType.DMA,
    ],
    compiler_params=pltpu.CompilerParams(needs_layout_passes=False),
)
def _gather_kernel(tableT_hbm, idx_hbm, out_hbm, idx_v, rlist_v, blist_v,
                   win_v, rows_v, wsem0, wsem1, osem):
    wid = lax.axis_index("s") * NC + lax.axis_index("c")
    c_lo = wid * 244 + jnp.minimum(wid, 5)
    c_hi = c_lo + 244 + jnp.where(wid < 5, 1, 0)
    r_lo = c_lo * LANE
    r_hi = jnp.minimum(c_hi * LANE, VOCAB)

    iota = lax.iota(jnp.int32, L)

    def fill(i, _):
        rlist_v[pl.ds(i * L, L)] = jnp.full((L,), SENT, jnp.int32)
        blist_v[pl.ds(i * L, L)] = jnp.full((L,), SENT, jnp.int32)
        return 0

    lax.fori_loop(0, LISTC // L, fill, 0, unroll=False)

    p = jnp.int32(0)
    for ch in range(NCHUNK):
        pltpu.sync_copy(idx_hbm.at[pl.ds(ch * CHUNK, CHUNK)], idx_v)

        def compact(q, pc):
            v = idx_v[pl.ds(q * L, L)]
            hit = (v >= r_lo) & (v < r_hi)
            hit_i = hit.astype(jnp.int32)
            pos = jnp.minimum(pc + plsc.cumsum(hit_i) - 1, LISTC - 1)
            bvec = iota + (ch * CHUNK + q * L)
            plsc.store_scatter(rlist_v, [pos], v, mask=hit)
            plsc.store_scatter(blist_v, [pos], bvec, mask=hit)
            n = plsc.all_reduce_population_count(hit)
            return pc + n[0]

        p = lax.fori_loop(0, CHUNK // L, compact, p, unroll=False)

    pgroups = (p + L - 1) // L

    LAST_START = (NCOL - WCOL) * LANE

    def win_start(e):
        return jnp.minimum((c_lo + WCOL * e) * LANE, LAST_START)

    def fetch(e, slot, sem):
        st = pl.multiple_of(win_start(e), LANE)
        pltpu.make_async_copy(
            tableT_hbm.at[:, pl.ds(st, WIN)],
            win_v.at[slot],
            sem,
        ).start()

    def drain_window(sem):
        pltpu.make_async_copy(
            tableT_hbm.at[:, pl.ds(0, WIN)], win_v.at[0], sem
        ).wait()

    def process(e, slot, nf0):
        st = win_start(e)

        def scan_group(q, nf):
            rvec = rlist_v[pl.ds(q * L, L)]
            in_ep = (rvec >= st) & (rvec < st + WIN)
            n = plsc.all_reduce_population_count(in_ep)[0]

            def do_group():
                bvec = blist_v[pl.ds(q * L, L)]
                ri_vec = rvec - st
                in_i = in_ep.astype(jnp.int32)
                for l in range(L):
                    hl = in_i[l] > 0

                    @pl.when(hl)
                    def _():
                        own = jnp.minimum(q * L + l, MAXH - 1)
                        row = own >> 1
                        col = pl.multiple_of((own & 1) * EMB, 8)
                        ri = jnp.full((L,), ri_vec[l], jnp.int32)
                        for k in range(EMB // L):
                            e_vec = iota + k * L
                            val = plsc.load_gather(win_v.at[slot], [e_vec, ri])
                            rows_v[row, pl.ds(col + k * L, L)] = val
                        boff = pl.multiple_of(bvec[l] * EMB, 8)
                        pltpu.make_async_copy(
                            rows_v.at[row, pl.ds(col, EMB)],
                            out_hbm.at[pl.ds(boff, EMB)],
                            osem,
                        ).start()

                return nf + n

            return lax.cond(n > 0, do_group, lambda: nf)

        return lax.fori_loop(0, pgroups, scan_group, nf0, unroll=False)

    fetch(0, 0, wsem0)

    def epoch_pair(i, nf):
        e0 = 2 * i
        fetch(e0 + 1, 1, wsem1)
        drain_window(wsem0)
        nf = process(e0, 0, nf)
        fetch(jnp.minimum(e0 + 2, NEP - 1), 0, wsem0)
        drain_window(wsem1)
        return process(e0 + 1, 1, nf)

    nfired = lax.fori_loop(0, NEP // 2, epoch_pair, jnp.int32(0),
                           unroll=False)
    drain_window(wsem0)

    def drain_rows(i, _):
        pltpu.make_async_copy(
            rows_v.at[0, pl.ds(0, EMB)], out_hbm.at[pl.ds(0, EMB)], osem
        ).wait()
        return 0

    lax.fori_loop(0, nfired, drain_rows, 0, unroll=False)


def kernel(input, hidden, cell, table):
    idx = input.astype(jnp.int32)
    out = _gather_kernel(table.T, idx)
    return out.reshape(B, 1, EMB)


# speedup vs baseline: 1.0625x; 1.0625x over previous
import functools

import jax
import jax.numpy as jnp
from jax import lax
from jax.experimental import pallas as pl
from jax.experimental.pallas import tpu as pltpu
from jax.experimental.pallas import tpu_sc as plsc

VOCAB = 1000000
EMB = 64
B = 16384
LANE = 128

_info = plsc.get_sparse_core_info()
NC, NS, L = _info.num_cores, _info.num_subcores, _info.num_lanes
NW = NC * NS
BPW = B // NW
SG = 2
NSG = L // SG


@functools.partial(
    pl.kernel,
    mesh=plsc.VectorSubcoreMesh(core_axis_name="c", subcore_axis_name="s"),
    out_type=jax.ShapeDtypeStruct((B, EMB), jnp.float32),
    scratch_types=[
        pltpu.VMEM((BPW,), jnp.int32),
        pltpu.VMEM((3, SG, EMB, LANE), jnp.float32),
        pltpu.VMEM((BPW, EMB), jnp.float32),
        pltpu.SemaphoreType.DMA,
    ],
    compiler_params=pltpu.CompilerParams(needs_layout_passes=False),
)
def _gather_kernel(tableT_hbm, idx_hbm, out_hbm, idx_v, win_v, rows_v, sem):
    wid = lax.axis_index("s") * NC + lax.axis_index("c")
    base = wid * BPW
    pltpu.sync_copy(idx_hbm.at[pl.ds(base, BPW)], idx_v)

    iota = lax.iota(jnp.int32, L)

    def body(g, _):
        vec = idx_v[pl.ds(g * L, L)]
        vbase = (vec >> 7) << 7
        ri_vec = vec & (LANE - 1)

        def fire(j):
            for l in range(SG):
                off = pl.multiple_of(vbase[j * SG + l], LANE)
                pltpu.make_async_copy(
                    tableT_hbm.at[:, pl.ds(off, LANE)],
                    win_v.at[j % 3, l],
                    sem,
                ).start()

        def drain():
            for l in range(SG):
                pltpu.make_async_copy(
                    tableT_hbm.at[:, pl.ds(0, LANE)],
                    win_v.at[0, l],
                    sem,
                ).wait()

        def extract(j):
            for l in range(SG):
                ri = jnp.full((L,), ri_vec[j * SG + l], jnp.int32)
                for k in range(EMB // L):
                    e_vec = iota + k * L
                    val = plsc.load_gather(win_v.at[j % 3, l], [e_vec, ri])
                    rows_v[g * L + j * SG + l, pl.ds(k * L, L)] = val

        fire(0)
        fire(1)
        for j in range(NSG):
            if j + 2 < NSG:
                fire(j + 2)
            drain()
            extract(j)
        return 0

    lax.fori_loop(0, BPW // L, body, 0, unroll=False)

    pltpu.sync_copy(rows_v, out_hbm.at[pl.ds(base, BPW)])


def kernel(input, hidden, cell, table):
    idx = input.astype(jnp.int32)
    out = _gather_kernel(table.T, idx)
    return out[:, None, :]
